# baseline (speedup 1.0000x reference)
import functools

import jax
import jax.numpy as jnp
import numpy as np
from jax import lax
from jax.experimental import pallas as pl
from jax.experimental.pallas import tpu as pltpu
from jax.experimental.pallas import tpu_sc as plsc

GAMMA = 12.0
EMB_INIT = 0.0546875
N_NODES = 10000
N_EDGES = 160000
D = 256

NC = 2
NS = 16
NW = NC * NS
NSLAB = 5
SLAB = N_EDGES // NSLAB
EPW = SLAB // NW
CHUNK = 40
NCHUNK = EPW // CHUNK


def _sc_gather(node_emb, src, dst):
    mesh = plsc.VectorSubcoreMesh(core_axis_name="c", subcore_axis_name="s")

    @functools.partial(
        pl.kernel,
        mesh=mesh,
        out_type=[
            jax.ShapeDtypeStruct((SLAB, D // 2), jnp.int32),
            jax.ShapeDtypeStruct((SLAB, D // 2), jnp.int32),
        ],
        scratch_types=[
            pltpu.VMEM((EPW,), jnp.int32),
            pltpu.VMEM((EPW,), jnp.int32),
            pltpu.VMEM((CHUNK, D // 2), jnp.int32),
            pltpu.VMEM((CHUNK, D // 2), jnp.int32),
            pltpu.VMEM((CHUNK, D // 2), jnp.int32),
            pltpu.VMEM((CHUNK, D // 2), jnp.int32),
            pltpu.SemaphoreType.DMA,
            pltpu.SemaphoreType.DMA,
            pltpu.SemaphoreType.DMA,
            pltpu.SemaphoreType.DMA,
        ],
    )
    def gather_kernel(node_hbm, src_hbm, dst_hbm, h_hbm, t_hbm,
                      sidx, didx, h0, t0, h1, t1, sg0, sg1, sw0, sw1):
        wid = lax.axis_index("s") * NC + lax.axis_index("c")
        base0 = wid * EPW
        pltpu.sync_copy(src_hbm.at[pl.ds(base0, EPW)], sidx)
        pltpu.sync_copy(dst_hbm.at[pl.ds(base0, EPW)], didx)

        hb, tb, sg, sw = [h0, h1], [t0, t1], [sg0, sg1], [sw0, sw1]
        g_pend = [None, None]
        w_pend = [None, None]
        for i in range(NCHUNK):
            b = i % 2
            off = i * CHUNK
            if w_pend[b] is not None:
                for cp in w_pend[b]:
                    cp.wait()
                w_pend[b] = None
            g_pend[b] = (
                pltpu.async_copy(
                    node_hbm.at[sidx.at[pl.ds(off, CHUNK)]], hb[b], sg[b]),
                pltpu.async_copy(
                    node_hbm.at[didx.at[pl.ds(off, CHUNK)]], tb[b], sg[b]),
            )
            pb = 1 - b
            if g_pend[pb] is not None:
                for cp in g_pend[pb]:
                    cp.wait()
                g_pend[pb] = None
                poff = base0 + (i - 1) * CHUNK
                w_pend[pb] = (
                    pltpu.async_copy(hb[pb], h_hbm.at[pl.ds(poff, CHUNK)], sw[pb]),
                    pltpu.async_copy(tb[pb], t_hbm.at[pl.ds(poff, CHUNK)], sw[pb]),
                )
        b = (NCHUNK - 1) % 2
        for cp in g_pend[b]:
            cp.wait()
        poff = base0 + (NCHUNK - 1) * CHUNK
        w_pend[b] = (
            pltpu.async_copy(hb[b], h_hbm.at[pl.ds(poff, CHUNK)], sw[b]),
            pltpu.async_copy(tb[b], t_hbm.at[pl.ds(poff, CHUNK)], sw[b]),
        )
        for b in (0, 1):
            if w_pend[b] is not None:
                for cp in w_pend[b]:
                    cp.wait()

    return gather_kernel(node_emb, src, dst)


_S1, _S3, _S5, _S7 = (np.float32(9.99997486e-01), np.float32(-1.66651677e-01),
                      np.float32(8.30951228e-03), np.float32(-1.84470858e-04))
_C0, _C2, _C4, _C6, _C8 = (np.float32(9.99999967e-01), np.float32(-4.99999269e-01),
                           np.float32(4.16640906e-02), np.float32(-1.38574158e-03),
                           np.float32(2.32374970e-05))
_PI_HI = np.float32(3.140625)
_PI_LO = np.float32(np.pi - 3.140625)


def _fast_sincos(phase):
    q = phase * np.float32(1.0 / np.pi)
    n = jnp.round(q)
    r = (phase - n * _PI_HI) - n * _PI_LO
    r2 = r * r
    s = r * (_S1 + r2 * (_S3 + r2 * (_S5 + r2 * _S7)))
    c = _C0 + r2 * (_C2 + r2 * (_C4 + r2 * (_C6 + r2 * _C8)))
    ni = n.astype(jnp.int32)
    mask = jax.lax.shift_left(ni, 31)
    s = jax.lax.bitcast_convert_type(
        jax.lax.bitcast_convert_type(s, jnp.int32) ^ mask, jnp.float32)
    c = jax.lax.bitcast_convert_type(
        jax.lax.bitcast_convert_type(c, jnp.int32) ^ mask, jnp.float32)
    return s, c


def _tc_score(h, t, edge_emb):
    B = 6400
    d = D // 2

    def unpack(w):
        re = jax.lax.bitcast_convert_type(jax.lax.shift_left(w, 16), jnp.float32)
        im = jax.lax.bitcast_convert_type(
            w & np.int32(np.uint32(0xFFFF0000)), jnp.float32)
        return re, im

    def body(h_ref, t_ref, e_ref, o_ref):
        hr, hi = unpack(h_ref[...])
        tr, ti = unpack(t_ref[...])
        ev = e_ref[...]
        delta = (ev[:, :d] - ev[:, d:]) * np.float32(np.pi / EMB_INIT)
        sd, cd = _fast_sincos(delta)
        u = hr * tr + hi * ti
        v = hi * tr - hr * ti
        norm = (hr * hr + hi * hi) + (tr * tr + ti * ti)
        sq = norm - 2.0 * (u * cd - v * sd)
        sc = jnp.sqrt(jnp.maximum(sq, 0.0))
        s = GAMMA - jnp.sum(sc, axis=1)
        o_ref[...] = s.reshape(1, B // 256, 256)

    out = pl.pallas_call(
        body,
        grid=(SLAB // B,),
        in_specs=[
            pl.BlockSpec((B, d), lambda i: (i, 0)),
            pl.BlockSpec((B, d), lambda i: (i, 0)),
            pl.BlockSpec((B, D), lambda i: (i, 0)),
        ],
        out_specs=pl.BlockSpec((1, B // 256, 256), lambda i: (i, 0, 0)),
        out_shape=jax.ShapeDtypeStruct(
            (SLAB // B, B // 256, 256), jnp.float32),
    )(h, t, edge_emb)
    return out.reshape(SLAB)


def kernel(node_emb, edge_emb, edge_index):
    src = edge_index[0]
    dst = edge_index[1]
    node_bf = node_emb.astype(jnp.bfloat16)
    node_i32 = jax.lax.bitcast_convert_type(
        jnp.stack([node_bf[:, :D // 2], node_bf[:, D // 2:]], axis=-1),
        jnp.int32)
    outs = []
    for s in range(NSLAB):
        lo = s * SLAB
        h, t = _sc_gather(node_i32, src[lo:lo + SLAB], dst[lo:lo + SLAB])
        outs.append(_tc_score(h, t, edge_emb[lo:lo + SLAB]))
    return jnp.concatenate(outs)


# speedup vs baseline: 1.0586x; 1.0586x over previous
import functools

import jax
import jax.numpy as jnp
import numpy as np
from jax import lax
from jax.experimental import pallas as pl
from jax.experimental.pallas import tpu as pltpu
from jax.experimental.pallas import tpu_sc as plsc

GAMMA = 12.0
EMB_INIT = 0.0546875
N_NODES = 10000
N_EDGES = 160000
D = 256

NC = 2
NS = 16
NW = NC * NS
NSLAB = 5
SLAB = N_EDGES // NSLAB
EPW = SLAB // NW
CHUNK = 40
NCHUNK = EPW // CHUNK


def _sc_gather(node_emb, src, dst):
    mesh = plsc.VectorSubcoreMesh(core_axis_name="c", subcore_axis_name="s")

    @functools.partial(
        pl.kernel,
        mesh=mesh,
        out_type=[
            jax.ShapeDtypeStruct((SLAB, D // 2), jnp.int32),
            jax.ShapeDtypeStruct((SLAB, D // 2), jnp.int32),
        ],
        scratch_types=[
            pltpu.VMEM((EPW,), jnp.int32),
            pltpu.VMEM((EPW,), jnp.int32),
            pltpu.VMEM((CHUNK, D // 2), jnp.int32),
            pltpu.VMEM((CHUNK, D // 2), jnp.int32),
            pltpu.SemaphoreType.DMA,
            pltpu.SemaphoreType.DMA,
        ],
    )
    def gather_kernel(node_hbm, src_hbm, dst_hbm, h_hbm, t_hbm,
                      sidx, didx, hrows, trows, sem_h, sem_t):
        wid = lax.axis_index("s") * NC + lax.axis_index("c")
        base0 = wid * EPW
        pltpu.sync_copy(src_hbm.at[pl.ds(base0, EPW)], sidx)
        pltpu.sync_copy(dst_hbm.at[pl.ds(base0, EPW)], didx)

        def body(i, carry):
            off = i * CHUNK
            cp_h = pltpu.async_copy(
                node_hbm.at[sidx.at[pl.ds(off, CHUNK)]], hrows, sem_h)
            cp_t = pltpu.async_copy(
                node_hbm.at[didx.at[pl.ds(off, CHUNK)]], trows, sem_t)
            cp_h.wait()
            cp_t.wait()
            pltpu.sync_copy(hrows, h_hbm.at[pl.ds(base0 + off, CHUNK)])
            pltpu.sync_copy(trows, t_hbm.at[pl.ds(base0 + off, CHUNK)])
            return carry

        lax.fori_loop(0, NCHUNK, body, 0)

    return gather_kernel(node_emb, src, dst)


_S1, _S3, _S5, _S7 = (np.float32(9.99997486e-01), np.float32(-1.66651677e-01),
                      np.float32(8.30951228e-03), np.float32(-1.84470858e-04))
_C0, _C2, _C4, _C6, _C8 = (np.float32(9.99999967e-01), np.float32(-4.99999269e-01),
                           np.float32(4.16640906e-02), np.float32(-1.38574158e-03),
                           np.float32(2.32374970e-05))
_PI_HI = np.float32(3.140625)
_PI_LO = np.float32(np.pi - 3.140625)


def _fast_sincos(phase):
    q = phase * np.float32(1.0 / np.pi)
    n = jnp.round(q)
    r = (phase - n * _PI_HI) - n * _PI_LO
    r2 = r * r
    s = r * (_S1 + r2 * (_S3 + r2 * (_S5 + r2 * _S7)))
    c = _C0 + r2 * (_C2 + r2 * (_C4 + r2 * (_C6 + r2 * _C8)))
    ni = n.astype(jnp.int32)
    mask = jax.lax.shift_left(ni, 31)
    s = jax.lax.bitcast_convert_type(
        jax.lax.bitcast_convert_type(s, jnp.int32) ^ mask, jnp.float32)
    c = jax.lax.bitcast_convert_type(
        jax.lax.bitcast_convert_type(c, jnp.int32) ^ mask, jnp.float32)
    return s, c


def _tc_score(h, t, edge_emb):
    B = 6400
    d = D // 2

    def unpack(w):
        re = jax.lax.bitcast_convert_type(jax.lax.shift_left(w, 16), jnp.float32)
        im = jax.lax.bitcast_convert_type(
            w & np.int32(np.uint32(0xFFFF0000)), jnp.float32)
        return re, im

    def body(h_ref, t_ref, e_ref, o_ref):
        hr, hi = unpack(h_ref[...])
        tr, ti = unpack(t_ref[...])
        ev = e_ref[...]
        delta = (ev[:, :d] - ev[:, d:]) * np.float32(np.pi / EMB_INIT)
        sd, cd = _fast_sincos(delta)
        u = hr * tr + hi * ti
        v = hi * tr - hr * ti
        norm = (hr * hr + hi * hi) + (tr * tr + ti * ti)
        sq = norm - 2.0 * (u * cd - v * sd)
        sc = jnp.sqrt(jnp.maximum(sq, 0.0))
        s = GAMMA - jnp.sum(sc, axis=1)
        o_ref[...] = s.reshape(1, B // 256, 256)

    out = pl.pallas_call(
        body,
        grid=(SLAB // B,),
        in_specs=[
            pl.BlockSpec((B, d), lambda i: (i, 0)),
            pl.BlockSpec((B, d), lambda i: (i, 0)),
            pl.BlockSpec((B, D), lambda i: (i, 0)),
        ],
        out_specs=pl.BlockSpec((1, B // 256, 256), lambda i: (i, 0, 0)),
        out_shape=jax.ShapeDtypeStruct(
            (SLAB // B, B // 256, 256), jnp.float32),
    )(h, t, edge_emb)
    return out.reshape(SLAB)


def kernel(node_emb, edge_emb, edge_index):
    src = edge_index[0]
    dst = edge_index[1]
    node_bf = node_emb.astype(jnp.bfloat16)
    node_i32 = jax.lax.bitcast_convert_type(
        jnp.stack([node_bf[:, :D // 2], node_bf[:, D // 2:]], axis=-1),
        jnp.int32)
    outs = []
    for s in range(NSLAB):
        lo = s * SLAB
        h, t = _sc_gather(node_i32, src[lo:lo + SLAB], dst[lo:lo + SLAB])
        outs.append(_tc_score(h, t, edge_emb[lo:lo + SLAB]))
    return jnp.concatenate(outs)
